# Initial kernel scaffold; baseline (speedup 1.0000x reference)
import jax
import jax.numpy as jnp
from jax.experimental import pallas as pl


def kernel(input_, weight):
    raise NotImplementedError("write your pallas kernel here")


# speedup vs baseline: 1.8473x; 1.8473x over previous
import functools

import jax
import jax.numpy as jnp
from jax import lax
from jax.experimental import pallas as pl
from jax.experimental.pallas import tpu as pltpu
from jax.experimental.pallas import tpu_sc as plsc

_D = 64
_B = 16384 * 50
_NC, _NS = 2, 16
_NW = _NC * _NS
_ROWS_PER_W = _B // _NW
_IDX_MINOR = 128
_DMAS_PER_GROUP = 8
_GROUP = _DMAS_PER_GROUP * _IDX_MINOR
_G = _ROWS_PER_W // _GROUP


@functools.partial(
    pl.kernel,
    out_type=jax.ShapeDtypeStruct((_B, _D), jnp.float32),
    mesh=plsc.VectorSubcoreMesh(core_axis_name="c", subcore_axis_name="s"),
    scratch_types=[
        pltpu.VMEM((_DMAS_PER_GROUP, _IDX_MINOR), jnp.int32),
        pltpu.VMEM((_GROUP, _D), jnp.float32),
        pltpu.SemaphoreType.DMA,
    ],
    compiler_params=pltpu.CompilerParams(use_tc_tiling_on_sc=False),
)
def _emb_gather(idx_hbm, table_hbm, out_hbm, idx_v, rows_v, sem):
    wid = lax.axis_index("s") * _NC + lax.axis_index("c")
    out_row0 = wid * _ROWS_PER_W
    idx_row0 = wid * (_ROWS_PER_W // _IDX_MINOR)

    def body(g, carry):
        pltpu.sync_copy(
            idx_hbm.at[pl.ds(idx_row0 + g * _DMAS_PER_GROUP, _DMAS_PER_GROUP)],
            idx_v,
        )
        handles = [
            pltpu.async_copy(
                table_hbm.at[idx_v.at[j]],
                rows_v.at[pl.ds(j * _IDX_MINOR, _IDX_MINOR)],
                sem,
            )
            for j in range(_DMAS_PER_GROUP)
        ]
        for h in handles:
            h.wait()
        pltpu.sync_copy(rows_v, out_hbm.at[pl.ds(out_row0 + g * _GROUP, _GROUP)])
        return carry

    lax.fori_loop(0, _G, body, 0)


def kernel(input_, weight):
    idx2d = input_.reshape(-1, _IDX_MINOR).astype(jnp.int32)
    out = _emb_gather(idx2d, weight)
    return out.reshape(input_.shape[0], input_.shape[1], _D)


# speedup vs baseline: 1.8640x; 1.0090x over previous
import functools

import jax
import jax.numpy as jnp
from jax import lax
from jax.experimental import pallas as pl
from jax.experimental.pallas import tpu as pltpu
from jax.experimental.pallas import tpu_sc as plsc

_D = 64
_B = 16384 * 50
_NC, _NS = 2, 16
_NW = _NC * _NS
_ROWS_PER_W = _B // _NW
_IDX_MINOR = 128
_DMAS_PER_GRP = 4
_GRP = _DMAS_PER_GRP * _IDX_MINOR
_PAIRS = _ROWS_PER_W // (2 * _GRP)


@functools.partial(
    pl.kernel,
    out_type=jax.ShapeDtypeStruct((_B, _D), jnp.float32),
    mesh=plsc.VectorSubcoreMesh(core_axis_name="c", subcore_axis_name="s"),
    scratch_types=[
        pltpu.VMEM((2 * _DMAS_PER_GRP, _IDX_MINOR), jnp.int32),
        pltpu.VMEM((_GRP, _D), jnp.float32),
        pltpu.VMEM((_GRP, _D), jnp.float32),
        pltpu.SemaphoreType.DMA,
        pltpu.SemaphoreType.DMA,
        pltpu.SemaphoreType.DMA,
        pltpu.SemaphoreType.DMA,
    ],
    compiler_params=pltpu.CompilerParams(use_tc_tiling_on_sc=False),
)
def _emb_gather(idx_hbm, table_hbm, out_hbm, idx_v, rows0, rows1,
                gsem0, gsem1, ssem0, ssem1):
    wid = lax.axis_index("s") * _NC + lax.axis_index("c")
    out_row0 = wid * _ROWS_PER_W
    idx_row0 = wid * (_ROWS_PER_W // _IDX_MINOR)
    rows = (rows0, rows1)
    gsem = (gsem0, gsem1)
    ssem = (ssem0, ssem1)

    def drain_store(b):
        pltpu.make_async_copy(
            rows[b], out_hbm.at[pl.ds(out_row0, _GRP)], ssem[b]
        ).wait()

    def body(p, carry):
        pltpu.sync_copy(idx_hbm.at[pl.ds(idx_row0 + p * 8, 8)], idx_v)
        handles = [[], []]
        for b in range(2):
            pl.when(p > 0)(lambda b=b: drain_store(b))
            for j in range(_DMAS_PER_GRP):
                handles[b].append(
                    pltpu.async_copy(
                        table_hbm.at[idx_v.at[b * _DMAS_PER_GRP + j]],
                        rows[b].at[pl.ds(j * _IDX_MINOR, _IDX_MINOR)],
                        gsem[b],
                    )
                )
        for b in range(2):
            for h in handles[b]:
                h.wait()
            pltpu.async_copy(
                rows[b],
                out_hbm.at[pl.ds(out_row0 + p * 2 * _GRP + b * _GRP, _GRP)],
                ssem[b],
            )
        return carry

    lax.fori_loop(0, _PAIRS, body, 0)
    for b in range(2):
        drain_store(b)


def kernel(input_, weight):
    idx2d = input_.reshape(-1, _IDX_MINOR).astype(jnp.int32)
    out = _emb_gather(idx2d, weight)
    return out.reshape(input_.shape[0], input_.shape[1], _D)
